# Initial kernel scaffold; baseline (speedup 1.0000x reference)
import jax
import jax.numpy as jnp
from jax.experimental import pallas as pl


def kernel(embeddings, labels, W):
    raise NotImplementedError("write your pallas kernel here")


# speedup vs baseline: 2.2106x; 2.2106x over previous
import functools

import jax
import jax.numpy as jnp
from jax import lax
from jax.experimental import pallas as pl
from jax.experimental.pallas import tpu as pltpu
from jax.experimental.pallas import tpu_sc as plsc

_NUM_CLASSES = 1000
_PAD_CLASSES = 1024
_TEMPERATURE = 0.05
_ROW_BLOCK = 512
_EPS = 1e-12


def _loss_block(emb_ref, w_ref, lab_ref, out_ref):
    i = pl.program_id(0)
    nblk = pl.num_programs(0)

    e = emb_ref[...]
    en = e / (jnp.sqrt(jnp.sum(e * e, axis=1, keepdims=True)) + _EPS)
    w = w_ref[...]
    wn = w / (jnp.sqrt(jnp.sum(w * w, axis=1, keepdims=True)) + _EPS)

    logits = lax.dot_general(
        en, wn, (((1,), (1,)), ((), ())), preferred_element_type=jnp.float32
    ) * (1.0 / _TEMPERATURE)

    col = lax.broadcasted_iota(jnp.int32, logits.shape, 1)
    logits = jnp.where(col < _NUM_CLASSES, logits, -1e30)

    m = jnp.max(logits, axis=1, keepdims=True)
    lse = m[:, 0] + jnp.log(jnp.sum(jnp.exp(logits - m), axis=1))

    lab = lab_ref[0, 0, :]
    picked = jnp.sum(jnp.where(col == lab[:, None], logits, 0.0), axis=1)

    part = jnp.sum(lse - picked)

    @pl.when(i == 0)
    def _init():
        out_ref[...] = jnp.zeros_like(out_ref)

    out_ref[...] += part.reshape(1, 1) * (1.0 / (nblk * _ROW_BLOCK))


def _loss_tc(embeddings, labels_i32, w_padded):
    b, d = embeddings.shape
    nblk = b // _ROW_BLOCK
    labs3 = labels_i32.reshape(nblk, 1, _ROW_BLOCK)
    out = pl.pallas_call(
        _loss_block,
        grid=(nblk,),
        in_specs=[
            pl.BlockSpec((_ROW_BLOCK, d), lambda i: (i, 0)),
            pl.BlockSpec((_PAD_CLASSES, d), lambda i: (0, 0)),
            pl.BlockSpec((1, 1, _ROW_BLOCK), lambda i: (i, 0, 0)),
        ],
        out_specs=pl.BlockSpec((1, 1), lambda i: (0, 0)),
        out_shape=jax.ShapeDtypeStruct((1, 1), jnp.float32),
    )(embeddings, w_padded, labs3)
    return out[0, 0]


def _labels_sc(labels_i32):
    b = labels_i32.shape[0]
    info = plsc.get_sparse_core_info()
    nw = info.num_cores * info.num_subcores
    per = b // nw
    mesh = plsc.VectorSubcoreMesh(core_axis_name="c", subcore_axis_name="s")

    @functools.partial(
        pl.kernel,
        mesh=mesh,
        out_type=jax.ShapeDtypeStruct((2 * b,), jnp.int32),
        scratch_types=[pltpu.VMEM((per,), jnp.int32)],
    )
    def k(lab_hbm, out_hbm, buf):
        wid = lax.axis_index("s") * info.num_cores + lax.axis_index("c")
        base = wid * per
        pltpu.sync_copy(lab_hbm.at[pl.ds(base, per)], buf)
        pltpu.sync_copy(buf, out_hbm.at[pl.ds(base, per)])
        pltpu.sync_copy(buf, out_hbm.at[pl.ds(b + base, per)])

    return k(labels_i32)


def kernel(embeddings, labels, W):
    labels_i32 = labels.astype(jnp.int32)
    w_padded = jnp.zeros((_PAD_CLASSES, W.shape[1]), W.dtype).at[:_NUM_CLASSES].set(W)
    loss = _loss_tc(embeddings, labels_i32, w_padded)
    combined_labels = _labels_sc(labels_i32).astype(labels.dtype)
    return (loss, combined_labels)


# speedup vs baseline: 2.4159x; 1.0929x over previous
import functools

import jax
import jax.numpy as jnp
from jax import lax
from jax.experimental import pallas as pl
from jax.experimental.pallas import tpu as pltpu
from jax.experimental.pallas import tpu_sc as plsc

_NUM_CLASSES = 1000
_PAD_CLASSES = 1024
_TEMPERATURE = 0.05
_ROW_BLOCK = 1024
_EPS = 1e-12


def _loss_block(emb_ref, w_ref, mask_ref, lab_ref, out_ref, wn_ref):
    i = pl.program_id(0)
    nblk = pl.num_programs(0)

    @pl.when(i == 0)
    def _prep():
        w = w_ref[...]
        wn_ref[...] = w * (
            (1.0 / _TEMPERATURE) / (jnp.sqrt(jnp.sum(w * w, axis=1, keepdims=True)) + _EPS)
        )
        out_ref[...] = jnp.zeros_like(out_ref)

    e = emb_ref[...]
    en = e / (jnp.sqrt(jnp.sum(e * e, axis=1, keepdims=True)) + _EPS)

    logits = lax.dot_general(
        en, wn_ref[...], (((1,), (1,)), ((), ())), preferred_element_type=jnp.float32
    )

    s = jnp.sum(jnp.exp(logits) * mask_ref[...], axis=1)
    lse = jnp.log(s)

    lab = lab_ref[0, 0, :]
    col = lax.broadcasted_iota(jnp.int32, logits.shape, 1)
    picked = jnp.sum(jnp.where(col == lab[:, None], logits, 0.0), axis=1)

    part = jnp.sum(lse - picked)
    out_ref[...] += part.reshape(1, 1) * (1.0 / (nblk * _ROW_BLOCK))


def _loss_tc(embeddings, labels_i32, w_padded, mask):
    b, d = embeddings.shape
    nblk = b // _ROW_BLOCK
    labs3 = labels_i32.reshape(nblk, 1, _ROW_BLOCK)
    out = pl.pallas_call(
        _loss_block,
        grid=(nblk,),
        in_specs=[
            pl.BlockSpec((_ROW_BLOCK, d), lambda i: (i, 0)),
            pl.BlockSpec((_PAD_CLASSES, d), lambda i: (0, 0)),
            pl.BlockSpec((1, _PAD_CLASSES), lambda i: (0, 0)),
            pl.BlockSpec((1, 1, _ROW_BLOCK), lambda i: (i, 0, 0)),
        ],
        out_specs=pl.BlockSpec((1, 1), lambda i: (0, 0)),
        out_shape=jax.ShapeDtypeStruct((1, 1), jnp.float32),
        scratch_shapes=[pltpu.VMEM((_PAD_CLASSES, d), jnp.float32)],
    )(embeddings, w_padded, mask, labs3)
    return out[0, 0]


def _labels_sc(labels_i32):
    b = labels_i32.shape[0]
    info = plsc.get_sparse_core_info()
    nw = info.num_cores * info.num_subcores
    per = b // nw
    mesh = plsc.VectorSubcoreMesh(core_axis_name="c", subcore_axis_name="s")

    @functools.partial(
        pl.kernel,
        mesh=mesh,
        out_type=jax.ShapeDtypeStruct((2 * b,), jnp.int32),
        scratch_types=[pltpu.VMEM((per,), jnp.int32)],
    )
    def k(lab_hbm, out_hbm, buf):
        wid = lax.axis_index("s") * info.num_cores + lax.axis_index("c")
        base = wid * per
        pltpu.sync_copy(lab_hbm.at[pl.ds(base, per)], buf)
        pltpu.sync_copy(buf, out_hbm.at[pl.ds(base, per)])
        pltpu.sync_copy(buf, out_hbm.at[pl.ds(b + base, per)])

    return k(labels_i32)


def kernel(embeddings, labels, W):
    labels_i32 = labels.astype(jnp.int32)
    w_padded = jnp.zeros((_PAD_CLASSES, W.shape[1]), W.dtype).at[:_NUM_CLASSES].set(W)
    mask = (
        jnp.arange(_PAD_CLASSES, dtype=jnp.int32) < _NUM_CLASSES
    ).astype(jnp.float32).reshape(1, _PAD_CLASSES)
    loss = _loss_tc(embeddings, labels_i32, w_padded, mask)
    combined_labels = _labels_sc(labels_i32).astype(labels.dtype)
    return (loss, combined_labels)


# speedup vs baseline: 2.5371x; 1.0502x over previous
import functools

import jax
import jax.numpy as jnp
from jax import lax
from jax.experimental import pallas as pl
from jax.experimental.pallas import tpu as pltpu
from jax.experimental.pallas import tpu_sc as plsc

_NUM_CLASSES = 1000
_PAD_CLASSES = 1024
_TEMPERATURE = 0.05
_ROW_BLOCK = 1024
_EPS = 1e-12


def _loss_block(emb_ref, w_ref, lab_ref, out_ref, wn_ref, mask_ref):
    i = pl.program_id(0)
    nblk = pl.num_programs(0)

    @pl.when(i == 0)
    def _prep():
        w = w_ref[...]
        wn_ref[: _NUM_CLASSES, :] = w * (
            (1.0 / _TEMPERATURE) / (jnp.sqrt(jnp.sum(w * w, axis=1, keepdims=True)) + _EPS)
        )
        wn_ref[_NUM_CLASSES:, :] = jnp.zeros(
            (_PAD_CLASSES - _NUM_CLASSES, w.shape[1]), jnp.float32
        )
        mcol = lax.broadcasted_iota(jnp.int32, (8, _PAD_CLASSES), 1)
        mask_ref[...] = (mcol < _NUM_CLASSES).astype(jnp.float32)
        out_ref[...] = jnp.zeros_like(out_ref)

    e = emb_ref[...]
    en = e / (jnp.sqrt(jnp.sum(e * e, axis=1, keepdims=True)) + _EPS)

    logits = lax.dot_general(
        en, wn_ref[...], (((1,), (1,)), ((), ())), preferred_element_type=jnp.float32
    )

    s = jnp.sum(jnp.exp(logits) * mask_ref[0:1, :], axis=1)
    lse = jnp.log(s)

    lab = lab_ref[0, 0, :]
    col = lax.broadcasted_iota(jnp.int32, logits.shape, 1)
    picked = jnp.sum(jnp.where(col == lab[:, None], logits, 0.0), axis=1)

    part = jnp.sum(lse - picked)
    out_ref[...] += part.reshape(1, 1) * (1.0 / (nblk * _ROW_BLOCK))


def _loss_tc(embeddings, labels_i32, W):
    b, d = embeddings.shape
    nblk = b // _ROW_BLOCK
    labs3 = labels_i32.reshape(nblk, 1, _ROW_BLOCK)
    out = pl.pallas_call(
        _loss_block,
        grid=(nblk,),
        in_specs=[
            pl.BlockSpec((_ROW_BLOCK, d), lambda i: (i, 0)),
            pl.BlockSpec((_NUM_CLASSES, d), lambda i: (0, 0)),
            pl.BlockSpec((1, 1, _ROW_BLOCK), lambda i: (i, 0, 0)),
        ],
        out_specs=pl.BlockSpec((1, 1), lambda i: (0, 0)),
        out_shape=jax.ShapeDtypeStruct((1, 1), jnp.float32),
        scratch_shapes=[
            pltpu.VMEM((_PAD_CLASSES, d), jnp.float32),
            pltpu.VMEM((8, _PAD_CLASSES), jnp.float32),
        ],
    )(embeddings, W, labs3)
    return out[0, 0]


def _labels_sc(labels_i32):
    b = labels_i32.shape[0]
    info = plsc.get_sparse_core_info()
    nw = info.num_cores * info.num_subcores
    per = b // nw
    mesh = plsc.VectorSubcoreMesh(core_axis_name="c", subcore_axis_name="s")

    @functools.partial(
        pl.kernel,
        mesh=mesh,
        out_type=jax.ShapeDtypeStruct((2 * b,), jnp.int32),
        scratch_types=[pltpu.VMEM((per,), jnp.int32)],
    )
    def k(lab_hbm, out_hbm, buf):
        wid = lax.axis_index("s") * info.num_cores + lax.axis_index("c")
        base = wid * per
        pltpu.sync_copy(lab_hbm.at[pl.ds(base, per)], buf)
        pltpu.sync_copy(buf, out_hbm.at[pl.ds(base, per)])
        pltpu.sync_copy(buf, out_hbm.at[pl.ds(b + base, per)])

    return k(labels_i32)


def kernel(embeddings, labels, W):
    labels_i32 = labels.astype(jnp.int32)
    loss = _loss_tc(embeddings, labels_i32, W)
    combined_labels = _labels_sc(labels_i32).astype(labels.dtype)
    return (loss, combined_labels)


# speedup vs baseline: 2.6506x; 1.0447x over previous
import functools

import jax
import jax.numpy as jnp
from jax import lax
from jax.experimental import pallas as pl
from jax.experimental.pallas import tpu as pltpu
from jax.experimental.pallas import tpu_sc as plsc

_NUM_CLASSES = 1000
_PAD_CLASSES = 1024
_TEMPERATURE = 0.05
_EPS = 1e-12


def _loss_block(emb_ref, w_ref, lab_ref, out_ref):
    w = w_ref[...]
    wn = jnp.concatenate(
        [
            w * (
                (1.0 / _TEMPERATURE)
                / (jnp.sqrt(jnp.sum(w * w, axis=1, keepdims=True)) + _EPS)
            ),
            jnp.zeros((_PAD_CLASSES - _NUM_CLASSES, w.shape[1]), jnp.float32),
        ],
        axis=0,
    )

    e = emb_ref[...]
    en = e / (jnp.sqrt(jnp.sum(e * e, axis=1, keepdims=True)) + _EPS)

    logits = lax.dot_general(
        en, wn, (((1,), (1,)), ((), ())), preferred_element_type=jnp.float32
    )

    col = lax.broadcasted_iota(jnp.int32, logits.shape, 1)
    mask = (col < _NUM_CLASSES).astype(jnp.float32)
    s = jnp.sum(jnp.exp(logits) * mask, axis=1)
    lse = jnp.log(s)

    lab = lab_ref[0, 0, :]
    picked = jnp.sum(jnp.where(col == lab[:, None], logits, 0.0), axis=1)

    loss = jnp.sum(lse - picked) * (1.0 / e.shape[0])
    out_ref[...] = loss.reshape(1, 1)


def _loss_tc(embeddings, labels_i32, W):
    b, d = embeddings.shape
    labs3 = labels_i32.reshape(1, 1, b)
    out = pl.pallas_call(
        _loss_block,
        in_specs=[
            pl.BlockSpec((b, d), lambda: (0, 0)),
            pl.BlockSpec((_NUM_CLASSES, d), lambda: (0, 0)),
            pl.BlockSpec((1, 1, b), lambda: (0, 0, 0)),
        ],
        out_specs=pl.BlockSpec((1, 1), lambda: (0, 0)),
        out_shape=jax.ShapeDtypeStruct((1, 1), jnp.float32),
    )(embeddings, W, labs3)
    return out[0, 0]


def _labels_sc(labels_i32):
    b = labels_i32.shape[0]
    info = plsc.get_sparse_core_info()
    nw = info.num_cores * info.num_subcores
    per = b // nw
    mesh = plsc.VectorSubcoreMesh(core_axis_name="c", subcore_axis_name="s")

    @functools.partial(
        pl.kernel,
        mesh=mesh,
        out_type=jax.ShapeDtypeStruct((2 * b,), jnp.int32),
        scratch_types=[pltpu.VMEM((per,), jnp.int32)],
    )
    def k(lab_hbm, out_hbm, buf):
        wid = lax.axis_index("s") * info.num_cores + lax.axis_index("c")
        base = wid * per
        pltpu.sync_copy(lab_hbm.at[pl.ds(base, per)], buf)
        pltpu.sync_copy(buf, out_hbm.at[pl.ds(base, per)])
        pltpu.sync_copy(buf, out_hbm.at[pl.ds(b + base, per)])

    return k(labels_i32)


def kernel(embeddings, labels, W):
    labels_i32 = labels.astype(jnp.int32)
    loss = _loss_tc(embeddings, labels_i32, W)
    combined_labels = _labels_sc(labels_i32).astype(labels.dtype)
    return (loss, combined_labels)


# speedup vs baseline: 5.6534x; 2.1329x over previous
import functools

import jax
import jax.numpy as jnp
from jax import lax
from jax.experimental import pallas as pl
from jax.experimental.pallas import tpu as pltpu
from jax.experimental.pallas import tpu_sc as plsc

_NUM_CLASSES = 1000
_PAD_CLASSES = 1024
_TEMPERATURE = 0.05
_EPS = 1e-12


def _loss_block(emb_ref, w_ref, lab_ref, out_ref):
    w = w_ref[...]
    wn = jnp.concatenate(
        [
            w * (
                (1.0 / _TEMPERATURE)
                / (jnp.sqrt(jnp.sum(w * w, axis=1, keepdims=True)) + _EPS)
            ),
            jnp.zeros((_PAD_CLASSES - _NUM_CLASSES, w.shape[1]), jnp.float32),
        ],
        axis=0,
    )

    e = emb_ref[...]
    en = e / (jnp.sqrt(jnp.sum(e * e, axis=1, keepdims=True)) + _EPS)

    logits = lax.dot_general(
        en, wn, (((1,), (1,)), ((), ())), preferred_element_type=jnp.float32
    )

    col = lax.broadcasted_iota(jnp.int32, logits.shape, 1)
    mask = (col < _NUM_CLASSES).astype(jnp.float32)
    s = jnp.sum(jnp.exp(logits) * mask, axis=1)
    lse = jnp.log(s)

    lab = lab_ref[0, 0, :]
    picked = jnp.sum(jnp.where(col == lab[:, None], logits, 0.0), axis=1)

    loss = jnp.sum(lse - picked) * (1.0 / e.shape[0])
    out_ref[...] = loss.reshape(1, 1)


def _loss_tc(embeddings, labels_i32, W):
    b, d = embeddings.shape
    labs3 = labels_i32.reshape(1, 1, b)
    out = pl.pallas_call(
        _loss_block,
        in_specs=[
            pl.BlockSpec((b, d), lambda: (0, 0)),
            pl.BlockSpec((_NUM_CLASSES, d), lambda: (0, 0)),
            pl.BlockSpec((1, 1, b), lambda: (0, 0, 0)),
        ],
        out_specs=pl.BlockSpec((1, 1), lambda: (0, 0)),
        out_shape=jax.ShapeDtypeStruct((1, 1), jnp.float32),
    )(embeddings, W, labs3)
    return out[0, 0]


def _labels_sc(labels_i32):
    b = labels_i32.shape[0]
    info = plsc.get_sparse_core_info()
    nw = info.num_cores * info.num_subcores
    per = b // nw
    mesh = plsc.VectorSubcoreMesh(core_axis_name="c", subcore_axis_name="s")

    @functools.partial(
        pl.kernel,
        mesh=mesh,
        out_type=jax.ShapeDtypeStruct((2 * b,), jnp.int32),
        scratch_types=[pltpu.VMEM((per,), jnp.int32)],
    )
    def k(lab_hbm, out_hbm, buf):
        wid = lax.axis_index("s") * info.num_cores + lax.axis_index("c")
        base = wid * per
        pltpu.sync_copy(lab_hbm.at[pl.ds(base, per)], buf)
        pltpu.sync_copy(buf, out_hbm.at[pl.ds(base, per)])
        pltpu.sync_copy(buf, out_hbm.at[pl.ds(b + base, per)])

    return k(labels_i32)


def kernel(embeddings, labels, W):
    labels_i32 = labels.astype(jnp.int32)
    loss = _loss_tc(embeddings, labels_i32, W)
    combined_labels = jnp.concatenate([labels_i32, labels_i32]).astype(labels.dtype)
    return (loss, combined_labels)


# speedup vs baseline: 10.1713x; 1.7991x over previous
import jax
import jax.numpy as jnp
from jax import lax
from jax.experimental import pallas as pl

_NUM_CLASSES = 1000
_PAD_CLASSES = 1024
_TEMPERATURE = 0.05
_EPS = 1e-12


def _loss_block(embT_ref, wT_ref, lab_ref, out_ref, cl_ref):
    wt = wT_ref[...]
    wn = jnp.concatenate(
        [
            wt * (
                (1.0 / _TEMPERATURE)
                / (jnp.sqrt(jnp.sum(wt * wt, axis=0, keepdims=True)) + _EPS)
            ),
            jnp.zeros((wt.shape[0], _PAD_CLASSES - _NUM_CLASSES), jnp.float32),
        ],
        axis=1,
    )

    et = embT_ref[...]
    en = et / (jnp.sqrt(jnp.sum(et * et, axis=0, keepdims=True)) + _EPS)

    logits = lax.dot_general(
        en, wn, (((0,), (0,)), ((), ())), preferred_element_type=jnp.float32
    )

    col = lax.broadcasted_iota(jnp.int32, logits.shape, 1)
    mask = (col < _NUM_CLASSES).astype(jnp.float32)
    s = jnp.sum(jnp.exp(logits) * mask, axis=1)
    lse = jnp.log(s)

    lab = lab_ref[0, 0, :]
    picked = jnp.sum(jnp.where(col == lab[:, None], logits, 0.0), axis=1)

    loss = jnp.sum(lse - picked) * (1.0 / logits.shape[0])
    out_ref[...] = loss.reshape(1, 1)

    b = lab.shape[0]
    cl_ref[0, pl.ds(0, b)] = lab
    cl_ref[0, pl.ds(b, b)] = lab


def _fused_tc(embeddings, labels_i32, W):
    b, d = embeddings.shape
    labs3 = labels_i32.reshape(1, 1, b)
    loss, cl = pl.pallas_call(
        _loss_block,
        in_specs=[
            pl.BlockSpec((d, b), lambda: (0, 0)),
            pl.BlockSpec((d, _NUM_CLASSES), lambda: (0, 0)),
            pl.BlockSpec((1, 1, b), lambda: (0, 0, 0)),
        ],
        out_specs=[
            pl.BlockSpec((1, 1), lambda: (0, 0)),
            pl.BlockSpec((1, 2 * b), lambda: (0, 0)),
        ],
        out_shape=[
            jax.ShapeDtypeStruct((1, 1), jnp.float32),
            jax.ShapeDtypeStruct((1, 2 * b), jnp.int32),
        ],
    )(embeddings.T, W.T, labs3)
    return loss[0, 0], cl.reshape(2 * b)


def kernel(embeddings, labels, W):
    labels_i32 = labels.astype(jnp.int32)
    loss, combined_labels = _fused_tc(embeddings, labels_i32, W)
    return (loss, combined_labels.astype(labels.dtype))


# speedup vs baseline: 11.2692x; 1.1079x over previous
import jax
import jax.numpy as jnp
from jax import lax
from jax.experimental import pallas as pl

_NUM_CLASSES = 1000
_PAD_CLASSES = 1024
_TEMPERATURE = 0.05
_EPS = 1e-12


_LOG2E = 1.4426950408889634
_LN2 = 0.6931471805599453


def _loss_block(embT_ref, wT_ref, lab_ref, out_ref, cl_ref):
    wt = wT_ref[...]
    d = wt.shape[0]
    wn = wt * (
        (_LOG2E / _TEMPERATURE)
        / (jnp.sqrt(jnp.sum(wt * wt, axis=0, keepdims=True)) + _EPS)
    )
    wn65 = jnp.concatenate([wn, jnp.zeros((1, _NUM_CLASSES), jnp.float32)], axis=0)
    pad_cols = jnp.concatenate(
        [
            jnp.zeros((d, _PAD_CLASSES - _NUM_CLASSES), jnp.float32),
            jnp.full((1, _PAD_CLASSES - _NUM_CLASSES), -1e30, jnp.float32),
        ],
        axis=0,
    )
    wfull = jnp.concatenate([wn65, pad_cols], axis=1)

    et = embT_ref[...]
    en = et / (jnp.sqrt(jnp.sum(et * et, axis=0, keepdims=True)) + _EPS)
    en65 = jnp.concatenate([en, jnp.ones((1, et.shape[1]), jnp.float32)], axis=0)

    lg = lax.dot_general(
        en65.astype(jnp.bfloat16),
        wfull.astype(jnp.bfloat16),
        (((0,), (0,)), ((), ())),
        preferred_element_type=jnp.float32,
    )

    s = jnp.sum(jnp.exp2(lg), axis=1)
    lse = jnp.log(s)

    lab = lab_ref[0, 0, :]
    col = lax.broadcasted_iota(jnp.int32, lg.shape, 1)
    picked2 = jnp.sum(jnp.where(col == lab[:, None], lg, 0.0), axis=1)

    loss = jnp.sum(lse - picked2 * _LN2) * (1.0 / lg.shape[0])
    out_ref[...] = loss.reshape(1, 1)

    b = lab.shape[0]
    cl_ref[0, pl.ds(0, b)] = lab
    cl_ref[0, pl.ds(b, b)] = lab


def _fused_tc(embeddings, labels_i32, W):
    b, d = embeddings.shape
    labs3 = labels_i32.reshape(1, 1, b)
    loss, cl = pl.pallas_call(
        _loss_block,
        in_specs=[
            pl.BlockSpec((d, b), lambda: (0, 0)),
            pl.BlockSpec((d, _NUM_CLASSES), lambda: (0, 0)),
            pl.BlockSpec((1, 1, b), lambda: (0, 0, 0)),
        ],
        out_specs=[
            pl.BlockSpec((1, 1), lambda: (0, 0)),
            pl.BlockSpec((1, 2 * b), lambda: (0, 0)),
        ],
        out_shape=[
            jax.ShapeDtypeStruct((1, 1), jnp.float32),
            jax.ShapeDtypeStruct((1, 2 * b), jnp.int32),
        ],
    )(embeddings.T, W.T, labs3)
    return loss[0, 0], cl.reshape(2 * b)


def kernel(embeddings, labels, W):
    labels_i32 = labels.astype(jnp.int32)
    loss, combined_labels = _fused_tc(embeddings, labels_i32, W)
    return (loss, combined_labels.astype(labels.dtype))
